# Initial kernel scaffold; baseline (speedup 1.0000x reference)
import jax
import jax.numpy as jnp
from jax.experimental import pallas as pl


def kernel(h, x, edge_index, params):
    raise NotImplementedError("write your pallas kernel here")


# speedup vs baseline: 2.5216x; 2.5216x over previous
import functools

import jax
import jax.numpy as jnp
from jax import lax
from jax.experimental import pallas as pl
from jax.experimental.pallas import tpu as pltpu
from jax.experimental.pallas import tpu_sc as plsc

F32 = jnp.float32

NC = 2
NS = 16
NW = NC * NS

BN = 512
BE = 512
CH = 128


def _silu(v):
    return v * jax.nn.sigmoid(v)


def _dot(a, b):
    return jnp.dot(a, b, preferred_element_type=F32)


def _bf(v):
    return v.astype(jnp.bfloat16).astype(F32)


def _embed_body(h_ref, eW_ref, eb_ref, h0_ref):
    h0_ref[...] = _dot(h_ref[...], eW_ref[...]) + eb_ref[...]


def _edge_body(E_real, hi_ref, hj_ref, xi_ref, xj_ref,
               W1_ref, b1_ref,
               W2_ref, b2_ref, cW1_ref, cb1_ref, cW2_ref, cb2_ref, wd2_ref,
               m_ref, w_ref):
    diff = xi_ref[...] - xj_ref[...]
    d2 = jnp.sum(diff * diff, axis=1, keepdims=True)
    hij = jnp.concatenate([hi_ref[...], hj_ref[...]], axis=1)
    pre = _dot(hij, W1_ref[...]) + _bf(d2) * _bf(wd2_ref[...]) + b1_ref[...]
    m = _silu(pre)
    m = _silu(_dot(m, W2_ref[...]) + b2_ref[...])
    c = _silu(_dot(m, cW1_ref[...]) + cb1_ref[...])
    cs = _dot(c, cW2_ref[...]) + cb2_ref[...]
    row = pl.program_id(0) * BE + lax.broadcasted_iota(jnp.int32, (BE, 1), 0)
    valid = (row < E_real).astype(F32)
    m_ref[...] = m * valid
    w_ref[...] = diff * (cs * valid)


def _post_body(inv_deg, h_ref, xp_ref, aggp_ref, xaccp_ref,
               nW1_ref, nb1_ref, nW2_ref, nb2_ref,
               hn_ref, xn_ref):
    h = h_ref[...]
    agg = aggp_ref[0] + aggp_ref[1]
    hu = jnp.concatenate([h, agg], axis=1)
    u = _silu(_dot(hu, nW1_ref[...]) + nb1_ref[...])
    u = _dot(u, nW2_ref[...]) + nb2_ref[...]
    hn_ref[...] = h + u
    xn_ref[...] = xp_ref[...] + (xaccp_ref[0] + xaccp_ref[1]) * inv_deg


def _post_final_body(inv_deg, h_ref, xp_ref, aggp_ref, xaccp_ref,
                     nW1_ref, nb1_ref, nW2_ref, nb2_ref,
                     oW_ref, ob_ref,
                     hout_ref, xn_ref):
    h = h_ref[...]
    agg = aggp_ref[0] + aggp_ref[1]
    hu = jnp.concatenate([h, agg], axis=1)
    u = _silu(_dot(hu, nW1_ref[...]) + nb1_ref[...])
    u = _dot(u, nW2_ref[...]) + nb2_ref[...]
    hn = h + u
    hout_ref[...] = _dot(hn, oW_ref[...]) + ob_ref[...]
    xn_ref[...] = xp_ref[...] + (xaccp_ref[0] + xaccp_ref[1]) * inv_deg


def _node_spec():
    return pl.BlockSpec((BN, 128), lambda i: (i, 0))


def _wspec(shape):
    nd = len(shape)
    return pl.BlockSpec(shape, lambda i, _n=nd: (0,) * _n)


def _part_spec():
    return pl.BlockSpec((2, BN, 128), lambda i: (0, i, 0))


def _part16_spec():
    return pl.BlockSpec((2, BN, 16), lambda i: (0, i, 0))


def _x_spec():
    return pl.BlockSpec((BN, 16), lambda i: (i, 0))


def _gather_body(CPW, A_hbm, B_hbm, xp_hbm, src_hbm, dst_hbm,
                 Ai_hbm, Bj_hbm, Xi_hbm, Xj_hbm,
                 idxs_v, idxd_v,
                 bufA0, bufB0, bufXi0, bufXj0,
                 bufA1, bufB1, bufXi1, bufXj1,
                 gsem0, gsem1, wsem0, wsem1):
    wid = lax.axis_index("s") * NC + lax.axis_index("c")
    base = wid * (CPW * CH)
    pltpu.sync_copy(dst_hbm.at[wid], idxd_v)
    pltpu.sync_copy(src_hbm.at[wid], idxs_v)

    bufs = ((bufA0, bufB0, bufXi0, bufXj0, gsem0, wsem0),
            (bufA1, bufB1, bufXi1, bufXj1, gsem1, wsem1))

    def g_descs(c, s):
        bA, bB, bXi, bXj, gs, _ = bufs[s]
        return ((A_hbm.at[idxd_v.at[c]], bA, gs),
                (B_hbm.at[idxs_v.at[c]], bB, gs),
                (xp_hbm.at[idxd_v.at[c]], bXi, gs),
                (xp_hbm.at[idxs_v.at[c]], bXj, gs))

    def w_descs(c, s):
        bA, bB, bXi, bXj, _, ws = bufs[s]
        off = base + c * CH
        return ((bA, Ai_hbm.at[pl.ds(off, CH)], ws),
                (bB, Bj_hbm.at[pl.ds(off, CH)], ws),
                (bXi, Xi_hbm.at[pl.ds(off, CH)], ws),
                (bXj, Xj_hbm.at[pl.ds(off, CH)], ws))

    def issue(descs):
        for sd in descs:
            pltpu.async_copy(*sd)

    def drain(descs):
        for sd in descs:
            pltpu.make_async_copy(*sd).wait()

    pairs = (CPW - 1) // 2
    issue(g_descs(0, 0))

    def body(i, carry):
        c0 = 2 * i
        c1 = c0 + 1
        drain(g_descs(c0, 0))
        issue(g_descs(c1, 1))
        issue(w_descs(c0, 0))
        drain(g_descs(c1, 1))
        drain(w_descs(c0, 0))
        issue(g_descs(c0 + 2, 0))
        issue(w_descs(c1, 1))
        drain(w_descs(c1, 1))
        return carry

    lax.fori_loop(0, pairs, body, 0)
    c_last = 2 * pairs
    drain(g_descs(c_last, 0))
    issue(w_descs(c_last, 0))
    drain(w_descs(c_last, 0))
    if CPW % 2 == 0:
        issue(g_descs(CPW - 1, 1))
        drain(g_descs(CPW - 1, 1))
        issue(w_descs(CPW - 1, 1))
        drain(w_descs(CPW - 1, 1))


def _scatter_body(CPW, N, m_hbm, w_hbm, dst_hbm, z128_hbm, z16_hbm,
                  aggp_hbm, xaccp_hbm,
                  bufm0, bufw0, idx0, bufm1, bufw1, idx1, agg_s, xacc_s,
                  rsem0, rsem1, ssem0, ssem1):
    cid = lax.axis_index("c")
    sid = lax.axis_index("s")
    wid = sid * NC + cid
    rpt = N // NS
    rbase = sid * rpt
    pltpu.sync_copy(z128_hbm.at[pl.ds(rbase, rpt)], agg_s.at[pl.ds(rbase, rpt)])
    pltpu.sync_copy(z16_hbm.at[pl.ds(rbase, rpt)], xacc_s.at[pl.ds(rbase, rpt)])
    plsc.subcore_barrier()

    base = wid * (CPW * CH)
    bufs = ((bufm0, bufw0, idx0, rsem0, ssem0),
            (bufm1, bufw1, idx1, rsem1, ssem1))

    def r_descs(c, s):
        bm, bw, ix, rs, _ = bufs[s]
        off = base + c * CH
        return ((m_hbm.at[pl.ds(off, CH)], bm, rs),
                (w_hbm.at[pl.ds(off, CH)], bw, rs),
                (dst_hbm.at[wid, pl.ds(c, 1)], ix, rs))

    def s_descs(c, s):
        bm, bw, ix, _, ss = bufs[s]
        return ((bm, agg_s.at[ix.at[0]], ss),
                (bw, xacc_s.at[ix.at[0]], ss))

    def issue(descs, add=False):
        for sd in descs:
            pltpu.async_copy(*sd, add=add)

    def drain(descs, add=False):
        for sd in descs:
            pltpu.make_async_copy(*sd).wait()

    pairs = (CPW - 1) // 2
    issue(r_descs(0, 0))

    def body(i, carry):
        c0 = 2 * i
        c1 = c0 + 1
        drain(r_descs(c0, 0))
        issue(r_descs(c1, 1))
        issue(s_descs(c0, 0), add=True)
        drain(r_descs(c1, 1))
        drain(s_descs(c0, 0))
        issue(r_descs(c0 + 2, 0))
        issue(s_descs(c1, 1), add=True)
        drain(s_descs(c1, 1))
        return carry

    lax.fori_loop(0, pairs, body, 0)
    c_last = 2 * pairs
    drain(r_descs(c_last, 0))
    issue(s_descs(c_last, 0), add=True)
    drain(s_descs(c_last, 0))
    if CPW % 2 == 0:
        issue(r_descs(CPW - 1, 1))
        drain(r_descs(CPW - 1, 1))
        issue(s_descs(CPW - 1, 1), add=True)
        drain(s_descs(CPW - 1, 1))
    plsc.subcore_barrier()
    pltpu.sync_copy(agg_s.at[pl.ds(rbase, rpt)],
                    aggp_hbm.at[cid, pl.ds(rbase, rpt)])
    pltpu.sync_copy(xacc_s.at[pl.ds(rbase, rpt)],
                    xaccp_hbm.at[cid, pl.ds(rbase, rpt)])


def kernel(h, x, edge_index, params):
    N, F = h.shape
    E = edge_index.shape[1]
    CPW = -(-E // (NW * CH))
    E2 = NW * CPW * CH
    EPW = CPW * CH
    GN = -(-N // BN)
    GE = E2 // BE
    inv_deg = 1.0 / float(E // N)

    src = jnp.pad(edge_index[0], (0, E2 - E)).reshape(NW, CPW, CH)
    dst = jnp.pad(edge_index[1], (0, E2 - E)).reshape(NW, CPW, CH)
    xpad = jnp.zeros((N, 16), F32).at[:, :3].set(x)
    z128 = jnp.zeros((N, 128), F32)
    z16 = jnp.zeros((N, 16), F32)

    mesh = plsc.VectorSubcoreMesh(core_axis_name="c", subcore_axis_name="s")
    sc_params = pltpu.CompilerParams(use_tc_tiling_on_sc=False)

    gather = pl.kernel(
        functools.partial(_gather_body, CPW),
        out_type=[
            jax.ShapeDtypeStruct((E2, 128), F32),
            jax.ShapeDtypeStruct((E2, 128), F32),
            jax.ShapeDtypeStruct((E2, 16), F32),
            jax.ShapeDtypeStruct((E2, 16), F32),
        ],
        mesh=mesh,
        scratch_types=[
            pltpu.VMEM((CPW, CH), jnp.int32),
            pltpu.VMEM((CPW, CH), jnp.int32),
            pltpu.VMEM((CH, 128), F32),
            pltpu.VMEM((CH, 128), F32),
            pltpu.VMEM((CH, 16), F32),
            pltpu.VMEM((CH, 16), F32),
            pltpu.VMEM((CH, 128), F32),
            pltpu.VMEM((CH, 128), F32),
            pltpu.VMEM((CH, 16), F32),
            pltpu.VMEM((CH, 16), F32),
            pltpu.SemaphoreType.DMA,
            pltpu.SemaphoreType.DMA,
            pltpu.SemaphoreType.DMA,
            pltpu.SemaphoreType.DMA,
        ],
        compiler_params=sc_params,
    )

    scatter = pl.kernel(
        functools.partial(_scatter_body, CPW, N),
        out_type=[
            jax.ShapeDtypeStruct((2, N, 128), F32),
            jax.ShapeDtypeStruct((2, N, 16), F32),
        ],
        mesh=mesh,
        scratch_types=[
            pltpu.VMEM((CH, 128), F32),
            pltpu.VMEM((CH, 16), F32),
            pltpu.VMEM((1, CH), jnp.int32),
            pltpu.VMEM((CH, 128), F32),
            pltpu.VMEM((CH, 16), F32),
            pltpu.VMEM((1, CH), jnp.int32),
            pltpu.VMEM_SHARED((N, 128), F32),
            pltpu.VMEM_SHARED((N, 16), F32),
            pltpu.SemaphoreType.DMA,
            pltpu.SemaphoreType.DMA,
            pltpu.SemaphoreType.DMA,
            pltpu.SemaphoreType.DMA,
        ],
        compiler_params=sc_params,
    )

    wmat = _wspec((128, 128))
    wcat = _wspec((256, 128))
    wrow = _wspec((1, 128))

    hcur = pl.pallas_call(
        _embed_body,
        grid=(GN,),
        in_specs=[_node_spec(), wmat, wrow],
        out_specs=_node_spec(),
        out_shape=jax.ShapeDtypeStruct((N, 128), F32),
    )(h, params['embed_in_W'], params['embed_in_b'].reshape(1, 128))

    edge_call = pl.pallas_call(
        functools.partial(_edge_body, E),
        grid=(GE,),
        in_specs=[
            pl.BlockSpec((BE, 128), lambda i: (i, 0)),
            pl.BlockSpec((BE, 128), lambda i: (i, 0)),
            pl.BlockSpec((BE, 16), lambda i: (i, 0)),
            pl.BlockSpec((BE, 16), lambda i: (i, 0)),
            wcat, wrow, wmat, wrow, wmat, wrow, _wspec((128, 1)),
            _wspec((1, 1)), wrow,
        ],
        out_specs=[
            pl.BlockSpec((BE, 128), lambda i: (i, 0)),
            pl.BlockSpec((BE, 16), lambda i: (i, 0)),
        ],
        out_shape=[
            jax.ShapeDtypeStruct((E2, 128), F32),
            jax.ShapeDtypeStruct((E2, 16), F32),
        ],
    )

    for li, lp in enumerate(params['layers']):
        Hi, Hj, Xi, Xj = gather(hcur, hcur, xpad, src, dst)
        m, w = edge_call(
            Hi, Hj, Xi, Xj,
            lp['edge_W1'][:256], lp['edge_b1'].reshape(1, 128),
            lp['edge_W2'], lp['edge_b2'].reshape(1, 128),
            lp['coord_W1'], lp['coord_b1'].reshape(1, 128),
            lp['coord_W2'], lp['coord_b2'].reshape(1, 1),
            lp['edge_W1'][256].reshape(1, 128))
        aggp, xaccp = scatter(m, w, dst, z128, z16)

        if li + 1 < len(params['layers']):
            hcur, xpad = pl.pallas_call(
                functools.partial(_post_body, inv_deg),
                grid=(GN,),
                in_specs=[_node_spec(), _x_spec(), _part_spec(), _part16_spec(),
                          wcat, wrow, wmat, wrow],
                out_specs=[_node_spec(), _x_spec()],
                out_shape=[
                    jax.ShapeDtypeStruct((N, 128), F32),
                    jax.ShapeDtypeStruct((N, 16), F32),
                ],
            )(hcur, xpad, aggp, xaccp,
              lp['node_W1'], lp['node_b1'].reshape(1, 128),
              lp['node_W2'], lp['node_b2'].reshape(1, 128))
        else:
            hout, xpad = pl.pallas_call(
                functools.partial(_post_final_body, inv_deg),
                grid=(GN,),
                in_specs=[_node_spec(), _x_spec(), _part_spec(), _part16_spec(),
                          wcat, wrow, wmat, wrow, wmat, wrow],
                out_specs=[_node_spec(), _x_spec()],
                out_shape=[
                    jax.ShapeDtypeStruct((N, 128), F32),
                    jax.ShapeDtypeStruct((N, 16), F32),
                ],
            )(hcur, xpad, aggp, xaccp,
              lp['node_W1'], lp['node_b1'].reshape(1, 128),
              lp['node_W2'], lp['node_b2'].reshape(1, 128),
              params['embed_out_W'], params['embed_out_b'].reshape(1, 128))

    return hout, xpad[:, :3]


# speedup vs baseline: 2.9387x; 1.1654x over previous
import functools

import jax
import jax.numpy as jnp
from jax import lax
from jax.experimental import pallas as pl
from jax.experimental.pallas import tpu as pltpu
from jax.experimental.pallas import tpu_sc as plsc

F32 = jnp.float32

NC = 2
NS = 16
NW = NC * NS

BN = 512
BE = 512
CH = 128


def _silu(v):
    return v * jax.nn.sigmoid(v)


def _dot(a, b):
    return jnp.dot(a, b, preferred_element_type=F32)


def _bf(v):
    return v.astype(jnp.bfloat16).astype(F32)


def _embed_body(h_ref, eW_ref, eb_ref, h0_ref):
    h0_ref[...] = _dot(h_ref[...], eW_ref[...]) + eb_ref[...]


def _edge_body(E_real, hi_ref, hj_ref, xi_ref, xj_ref,
               W1_ref, b1_ref,
               W2_ref, b2_ref, cW1_ref, cb1_ref, cW2_ref, cb2_ref, wd2_ref,
               m_ref, w_ref):
    diff = xi_ref[...] - xj_ref[...]
    d2 = jnp.sum(diff * diff, axis=1, keepdims=True)
    hij = jnp.concatenate([hi_ref[...], hj_ref[...]], axis=1)
    pre = _dot(hij, W1_ref[...]) + _bf(d2) * _bf(wd2_ref[...]) + b1_ref[...]
    m = _silu(pre)
    m = _silu(_dot(m, W2_ref[...]) + b2_ref[...])
    c = _silu(_dot(m, cW1_ref[...]) + cb1_ref[...])
    cs = _dot(c, cW2_ref[...]) + cb2_ref[...]
    row = pl.program_id(0) * BE + lax.broadcasted_iota(jnp.int32, (BE, 1), 0)
    valid = (row < E_real).astype(F32)
    m_ref[...] = m * valid
    w_ref[...] = diff * (cs * valid)


def _post_body(inv_deg, h_ref, xp_ref, aggpA_ref, aggpB_ref,
               xaccpA_ref, xaccpB_ref,
               nW1_ref, nb1_ref, nW2_ref, nb2_ref,
               hn_ref, xn_ref):
    h = h_ref[...]
    agg = (aggpA_ref[0] + aggpA_ref[1]) + (aggpB_ref[0] + aggpB_ref[1])
    hu = jnp.concatenate([h, agg], axis=1)
    u = _silu(_dot(hu, nW1_ref[...]) + nb1_ref[...])
    u = _dot(u, nW2_ref[...]) + nb2_ref[...]
    hn_ref[...] = h + u
    xacc = (xaccpA_ref[0] + xaccpA_ref[1]) + (xaccpB_ref[0] + xaccpB_ref[1])
    xn_ref[...] = xp_ref[...] + xacc * inv_deg


def _post_final_body(inv_deg, h_ref, xp_ref, aggpA_ref, aggpB_ref,
                     xaccpA_ref, xaccpB_ref,
                     nW1_ref, nb1_ref, nW2_ref, nb2_ref,
                     oW_ref, ob_ref,
                     hout_ref, xn_ref):
    h = h_ref[...]
    agg = (aggpA_ref[0] + aggpA_ref[1]) + (aggpB_ref[0] + aggpB_ref[1])
    hu = jnp.concatenate([h, agg], axis=1)
    u = _silu(_dot(hu, nW1_ref[...]) + nb1_ref[...])
    u = _dot(u, nW2_ref[...]) + nb2_ref[...]
    hn = h + u
    hout_ref[...] = _dot(hn, oW_ref[...]) + ob_ref[...]
    xacc = (xaccpA_ref[0] + xaccpA_ref[1]) + (xaccpB_ref[0] + xaccpB_ref[1])
    xn_ref[...] = xp_ref[...] + xacc * inv_deg


def _node_spec():
    return pl.BlockSpec((BN, 128), lambda i: (i, 0))


def _wspec(shape):
    nd = len(shape)
    return pl.BlockSpec(shape, lambda i, _n=nd: (0,) * _n)


def _part_spec():
    return pl.BlockSpec((2, BN, 128), lambda i: (0, i, 0))


def _part16_spec():
    return pl.BlockSpec((2, BN, 16), lambda i: (0, i, 0))


def _x_spec():
    return pl.BlockSpec((BN, 16), lambda i: (i, 0))


CH2 = 384


def _gather_body(EPW, A_hbm, B_hbm, xp_hbm, src_hbm, dst_hbm,
                 Ai_hbm, Bj_hbm, Xi_hbm, Xj_hbm,
                 idxs_v, idxd_v, bufA, bufB, bufXi, bufXj,
                 semA, semB, semXi, semXj):
    wid = lax.axis_index("s") * NC + lax.axis_index("c")
    base = wid * EPW
    pltpu.sync_copy(dst_hbm.at[wid], idxd_v)
    pltpu.sync_copy(src_hbm.at[wid], idxs_v)

    n_full = EPW // CH2
    tail = EPW - n_full * CH2

    def descs(off, size):
        ds = pl.ds(off, size)
        bs = pl.ds(0, size)
        od = pl.ds(base + off, size)
        return (
            (A_hbm.at[idxd_v.at[ds]], bufA.at[bs], semA, Ai_hbm.at[od]),
            (B_hbm.at[idxs_v.at[ds]], bufB.at[bs], semB, Bj_hbm.at[od]),
            (xp_hbm.at[idxd_v.at[ds]], bufXi.at[bs], semXi, Xi_hbm.at[od]),
            (xp_hbm.at[idxs_v.at[ds]], bufXj.at[bs], semXj, Xj_hbm.at[od]),
        )

    def run_chunk(off, size):
        dd = descs(off, size)
        for gsrc, buf, sem, _ in dd:
            pltpu.async_copy(gsrc, buf, sem)
        for gsrc, buf, sem, out in dd:
            pltpu.make_async_copy(gsrc, buf, sem).wait()
            pltpu.async_copy(buf, out, sem)
        for gsrc, buf, sem, out in dd:
            pltpu.make_async_copy(buf, out, sem).wait()

    def body(c, carry):
        run_chunk(c * CH2, CH2)
        return carry

    lax.fori_loop(0, n_full, body, 0)
    if tail:
        run_chunk(n_full * CH2, tail)


def _scatter_body(CPW, N, m_hbm, w_hbm, dst_hbm, z128_hbm, z16_hbm,
                  aggp_hbm, xaccp_hbm,
                  bufm0, bufw0, idx0, bufm1, bufw1, idx1, agg_s, xacc_s,
                  rsem0, rsem1, ssem0, ssem1):
    cid = lax.axis_index("c")
    sid = lax.axis_index("s")
    wid = sid * NC + cid
    rpt = N // NS
    rbase = sid * rpt
    pltpu.sync_copy(z128_hbm.at[pl.ds(rbase, rpt)], agg_s.at[pl.ds(rbase, rpt)])
    pltpu.sync_copy(z16_hbm.at[pl.ds(rbase, rpt)], xacc_s.at[pl.ds(rbase, rpt)])
    plsc.subcore_barrier()

    base = wid * (CPW * CH)
    bufs = ((bufm0, bufw0, idx0, rsem0, ssem0),
            (bufm1, bufw1, idx1, rsem1, ssem1))

    def r_descs(c, s):
        bm, bw, ix, rs, _ = bufs[s]
        off = base + c * CH
        return ((m_hbm.at[pl.ds(off, CH)], bm, rs),
                (w_hbm.at[pl.ds(off, CH)], bw, rs),
                (dst_hbm.at[wid, pl.ds(c, 1)], ix, rs))

    def s_descs(c, s):
        bm, bw, ix, _, ss = bufs[s]
        return ((bm, agg_s.at[ix.at[0]], ss),
                (bw, xacc_s.at[ix.at[0]], ss))

    def issue(descs, add=False):
        for sd in descs:
            pltpu.async_copy(*sd, add=add)

    def drain(descs, add=False):
        for sd in descs:
            pltpu.make_async_copy(*sd).wait()

    pairs = (CPW - 1) // 2
    issue(r_descs(0, 0))

    def body(i, carry):
        c0 = 2 * i
        c1 = c0 + 1
        drain(r_descs(c0, 0))
        issue(r_descs(c1, 1))
        issue(s_descs(c0, 0), add=True)
        drain(r_descs(c1, 1))
        drain(s_descs(c0, 0))
        issue(r_descs(c0 + 2, 0))
        issue(s_descs(c1, 1), add=True)
        drain(s_descs(c1, 1))
        return carry

    lax.fori_loop(0, pairs, body, 0)
    c_last = 2 * pairs
    drain(r_descs(c_last, 0))
    issue(s_descs(c_last, 0), add=True)
    drain(s_descs(c_last, 0))
    if CPW % 2 == 0:
        issue(r_descs(CPW - 1, 1))
        drain(r_descs(CPW - 1, 1))
        issue(s_descs(CPW - 1, 1), add=True)
        drain(s_descs(CPW - 1, 1))
    plsc.subcore_barrier()
    pltpu.sync_copy(agg_s.at[pl.ds(rbase, rpt)],
                    aggp_hbm.at[cid, pl.ds(rbase, rpt)])
    pltpu.sync_copy(xacc_s.at[pl.ds(rbase, rpt)],
                    xaccp_hbm.at[cid, pl.ds(rbase, rpt)])


def kernel(h, x, edge_index, params):
    N, F = h.shape
    E = edge_index.shape[1]
    CPW = -(-E // (NW * CH))
    E2 = NW * CPW * CH
    EPW = CPW * CH
    GN = -(-N // BN)
    inv_deg = 1.0 / float(E // N)

    CPW_A = (CPW + 1) // 2
    CPW_B = CPW - CPW_A
    halves = []
    src_flat = jnp.pad(edge_index[0], (0, E2 - E)).reshape(NW, CPW, CH)
    dst_flat = jnp.pad(edge_index[1], (0, E2 - E)).reshape(NW, CPW, CH)
    last_valid = E - (NW - 1) * EPW
    for cpw_h, c_lo in ((CPW_A, 0), (CPW_B, CPW_A)):
        eph = cpw_h * CH
        srch = src_flat[:, c_lo:c_lo + cpw_h]
        dsth = dst_flat[:, c_lo:c_lo + cpw_h]
        vh = ((NW - 1) * eph
              + max(0, min(eph, last_valid - c_lo * CH)))
        halves.append(dict(cpw=cpw_h, eph=eph, eh=NW * eph,
                           src2=srch.reshape(NW, eph), dst3=dsth,
                           dst2=dsth.reshape(NW, eph), valid=vh))

    xpad = jnp.zeros((N, 16), F32).at[:, :3].set(x)
    z128 = jnp.zeros((N, 128), F32)
    z16 = jnp.zeros((N, 16), F32)

    mesh = plsc.VectorSubcoreMesh(core_axis_name="c", subcore_axis_name="s")
    sc_params = pltpu.CompilerParams(use_tc_tiling_on_sc=False)

    def make_gather(hv):
        return pl.kernel(
            functools.partial(_gather_body, hv['eph']),
            out_type=[
                jax.ShapeDtypeStruct((hv['eh'], 128), F32),
                jax.ShapeDtypeStruct((hv['eh'], 128), F32),
                jax.ShapeDtypeStruct((hv['eh'], 16), F32),
                jax.ShapeDtypeStruct((hv['eh'], 16), F32),
            ],
            mesh=mesh,
            scratch_types=[
                pltpu.VMEM((hv['eph'],), jnp.int32),
                pltpu.VMEM((hv['eph'],), jnp.int32),
                pltpu.VMEM((CH2, 128), F32),
                pltpu.VMEM((CH2, 128), F32),
                pltpu.VMEM((CH2, 16), F32),
                pltpu.VMEM((CH2, 16), F32),
                pltpu.SemaphoreType.DMA,
                pltpu.SemaphoreType.DMA,
                pltpu.SemaphoreType.DMA,
                pltpu.SemaphoreType.DMA,
            ],
            compiler_params=sc_params,
        )

    def make_scatter(hv):
        return pl.kernel(
            functools.partial(_scatter_body, hv['cpw'], N),
            out_type=[
                jax.ShapeDtypeStruct((2, N, 128), F32),
                jax.ShapeDtypeStruct((2, N, 16), F32),
            ],
            mesh=mesh,
            scratch_types=[
                pltpu.VMEM((CH, 128), F32),
                pltpu.VMEM((CH, 16), F32),
                pltpu.VMEM((1, CH), jnp.int32),
                pltpu.VMEM((CH, 128), F32),
                pltpu.VMEM((CH, 16), F32),
                pltpu.VMEM((1, CH), jnp.int32),
                pltpu.VMEM_SHARED((N, 128), F32),
                pltpu.VMEM_SHARED((N, 16), F32),
                pltpu.SemaphoreType.DMA,
                pltpu.SemaphoreType.DMA,
                pltpu.SemaphoreType.DMA,
                pltpu.SemaphoreType.DMA,
            ],
            compiler_params=sc_params,
        )

    gathers = [make_gather(hv) for hv in halves]
    scatters = [make_scatter(hv) for hv in halves]

    wmat = _wspec((128, 128))
    wcat = _wspec((256, 128))
    wrow = _wspec((1, 128))

    hcur = pl.pallas_call(
        _embed_body,
        grid=(GN,),
        in_specs=[_node_spec(), wmat, wrow],
        out_specs=_node_spec(),
        out_shape=jax.ShapeDtypeStruct((N, 128), F32),
    )(h, params['embed_in_W'], params['embed_in_b'].reshape(1, 128))

    def make_edge(hv):
        return pl.pallas_call(
            functools.partial(_edge_body, hv['valid']),
            grid=(hv['eh'] // BE,),
            in_specs=[
                pl.BlockSpec((BE, 128), lambda i: (i, 0)),
                pl.BlockSpec((BE, 128), lambda i: (i, 0)),
                pl.BlockSpec((BE, 16), lambda i: (i, 0)),
                pl.BlockSpec((BE, 16), lambda i: (i, 0)),
                wcat, wrow, wmat, wrow, wmat, wrow, _wspec((128, 1)),
                _wspec((1, 1)), wrow,
            ],
            out_specs=[
                pl.BlockSpec((BE, 128), lambda i: (i, 0)),
                pl.BlockSpec((BE, 16), lambda i: (i, 0)),
            ],
            out_shape=[
                jax.ShapeDtypeStruct((hv['eh'], 128), F32),
                jax.ShapeDtypeStruct((hv['eh'], 16), F32),
            ],
        )

    edges = [make_edge(hv) for hv in halves]

    for li, lp in enumerate(params['layers']):
        ew = (lp['edge_W1'][:256], lp['edge_b1'].reshape(1, 128),
              lp['edge_W2'], lp['edge_b2'].reshape(1, 128),
              lp['coord_W1'], lp['coord_b1'].reshape(1, 128),
              lp['coord_W2'], lp['coord_b2'].reshape(1, 1),
              lp['edge_W1'][256].reshape(1, 128))
        g = [gathers[s](hcur, hcur, xpad, halves[s]['src2'],
                        halves[s]['dst2']) for s in range(2)]
        mwA = edges[0](*g[0], *ew)
        aggpA, xaccpA = scatters[0](*mwA, halves[0]['dst3'], z128, z16)
        mwB = edges[1](*g[1], *ew)
        aggpB, xaccpB = scatters[1](*mwB, halves[1]['dst3'], z128, z16)

        if li + 1 < len(params['layers']):
            hcur, xpad = pl.pallas_call(
                functools.partial(_post_body, inv_deg),
                grid=(GN,),
                in_specs=[_node_spec(), _x_spec(),
                          _part_spec(), _part_spec(),
                          _part16_spec(), _part16_spec(),
                          wcat, wrow, wmat, wrow],
                out_specs=[_node_spec(), _x_spec()],
                out_shape=[
                    jax.ShapeDtypeStruct((N, 128), F32),
                    jax.ShapeDtypeStruct((N, 16), F32),
                ],
            )(hcur, xpad, aggpA, aggpB, xaccpA, xaccpB,
              lp['node_W1'], lp['node_b1'].reshape(1, 128),
              lp['node_W2'], lp['node_b2'].reshape(1, 128))
        else:
            hout, xpad = pl.pallas_call(
                functools.partial(_post_final_body, inv_deg),
                grid=(GN,),
                in_specs=[_node_spec(), _x_spec(),
                          _part_spec(), _part_spec(),
                          _part16_spec(), _part16_spec(),
                          wcat, wrow, wmat, wrow, wmat, wrow],
                out_specs=[_node_spec(), _x_spec()],
                out_shape=[
                    jax.ShapeDtypeStruct((N, 128), F32),
                    jax.ShapeDtypeStruct((N, 16), F32),
                ],
            )(hcur, xpad, aggpA, aggpB, xaccpA, xaccpB,
              lp['node_W1'], lp['node_b1'].reshape(1, 128),
              lp['node_W2'], lp['node_b2'].reshape(1, 128),
              params['embed_out_W'], params['embed_out_b'].reshape(1, 128))

    return hout, xpad[:, :3]
